# Initial kernel scaffold; baseline (speedup 1.0000x reference)
import functools

import jax
import jax.numpy as jnp
from jax import lax
from jax.experimental import pallas as pl
from jax.experimental.pallas import tpu as pltpu
from jax.experimental.pallas import tpu_sc as plsc

NC = 2
NS = 16
L = 16


def _make_kernel(B, S, V, E):
    NW = NC * NS
    assert B % NW == 0
    IPW = B // NW

    chunks = []
    off = 0
    while off < S:
        n = min(128, S - off)
        chunks.append((off, n))
        off += n

    mesh = plsc.VectorSubcoreMesh(
        core_axis_name="c", subcore_axis_name="s", num_cores=NC, num_subcores=NS
    )

    @functools.partial(
        pl.kernel,
        out_type=jax.ShapeDtypeStruct((B, E, S), jnp.float32),
        mesh=mesh,
        scratch_types=[
            pltpu.VMEM((IPW, S), jnp.int32),
            pltpu.VMEM((S, E), jnp.float32),
            pltpu.VMEM((E, S), jnp.float32),
            pltpu.SemaphoreType.DMA,
        ],
    )
    def k(inputs_hbm, table_hbm, out_hbm, idx_v, rows_v, out_v, sem):
        wid = lax.axis_index("s") * NC + lax.axis_index("c")
        base_b = wid * IPW

        pltpu.sync_copy(inputs_hbm.at[pl.ds(base_b, IPW), :], idx_v)

        eidx = [jnp.arange(L, dtype=jnp.int32) + j * L for j in range(E // L)]

        def item_body(item, carry):
            for (o, n) in chunks:
                pltpu.async_copy(
                    table_hbm.at[idx_v.at[item, pl.ds(o, n)]],
                    rows_v.at[pl.ds(o, n), :],
                    sem,
                ).wait()

            def s_body(s, c):
                sv = jnp.full((L,), s, dtype=jnp.int32)
                for j in range(E // L):
                    v = rows_v[s, pl.ds(j * L, L)]
                    plsc.store_scatter(out_v, [eidx[j], sv], v)
                return c

            lax.fori_loop(0, S, s_body, None)

            pltpu.sync_copy(out_v, out_hbm.at[base_b + item])
            return carry

        lax.fori_loop(0, IPW, item_body, None)

    return k


def kernel(inputs, table):
    B, S = inputs.shape
    V, E = table.shape
    inputs = inputs.astype(jnp.int32)
    k = _make_kernel(B, S, V, E)
    return k(inputs, table)


# speedup vs baseline: 2.2713x; 2.2713x over previous
import functools

import jax
import jax.numpy as jnp
from jax import lax
from jax.experimental import pallas as pl
from jax.experimental.pallas import tpu as pltpu
from jax.experimental.pallas import tpu_sc as plsc

NC = 2
NS = 16
L = 16


def _make_kernel(B, S, V, E):
    NW = NC * NS
    assert B % NW == 0
    IPW = B // NW

    chunks = []
    off = 0
    while off < S:
        n = min(128, S - off)
        chunks.append((off, n))
        off += n

    mesh = plsc.VectorSubcoreMesh(
        core_axis_name="c", subcore_axis_name="s", num_cores=NC, num_subcores=NS
    )

    @functools.partial(
        pl.kernel,
        out_type=jax.ShapeDtypeStruct((B, E, S), jnp.float32),
        mesh=mesh,
        scratch_types=[
            pltpu.VMEM((IPW, S), jnp.int32),
            pltpu.VMEM((S, E), jnp.float32),
            pltpu.VMEM((E, S), jnp.float32),
            pltpu.SemaphoreType.DMA,
        ],
        compiler_params=pltpu.CompilerParams(
            use_tc_tiling_on_sc=False, needs_layout_passes=False
        ),
    )
    def k(inputs_hbm, table_hbm, out_hbm, idx_v, rows_v, out_v, sem):
        wid = lax.axis_index("s") * NC + lax.axis_index("c")
        base_b = wid * IPW

        pltpu.sync_copy(inputs_hbm.at[pl.ds(base_b, IPW), :], idx_v)

        eidx = [jnp.arange(L, dtype=jnp.int32) + j * L for j in range(E // L)]

        def item_body(item, carry):
            for (o, n) in chunks:
                pltpu.async_copy(
                    table_hbm.at[idx_v.at[item, pl.ds(o, n)]],
                    rows_v.at[pl.ds(o, n), :],
                    sem,
                ).wait()

            def s_body(s, c):
                sv = jnp.full((L,), s, dtype=jnp.int32)
                for j in range(E // L):
                    v = rows_v[s, pl.ds(j * L, L)]
                    plsc.store_scatter(out_v, [eidx[j], sv], v)
                return c

            lax.fori_loop(0, S, s_body, None)

            pltpu.sync_copy(out_v, out_hbm.at[base_b + item])
            return carry

        lax.fori_loop(0, IPW, item_body, None)

    return k


def kernel(inputs, table):
    B, S = inputs.shape
    V, E = table.shape
    inputs = inputs.astype(jnp.int32)
    k = _make_kernel(B, S, V, E)
    return k(inputs, table)


# speedup vs baseline: 2.5471x; 1.1214x over previous
import functools

import jax
import jax.numpy as jnp
from jax import lax
from jax.experimental import pallas as pl
from jax.experimental.pallas import tpu as pltpu
from jax.experimental.pallas import tpu_sc as plsc

NC = 2
NS = 16
L = 16
SUNROLL = 4


def _make_kernel(B, S, V, E):
    NW = NC * NS
    assert B % NW == 0 and B // NW % 2 == 0 and S % SUNROLL == 0
    IPW = B // NW
    OP = S + 1

    chunks = []
    off = 0
    while off < S:
        n = min(128, S - off)
        chunks.append((off, n))
        off += n

    mesh = plsc.VectorSubcoreMesh(
        core_axis_name="c", subcore_axis_name="s", num_cores=NC, num_subcores=NS
    )

    @functools.partial(
        pl.kernel,
        out_type=jax.ShapeDtypeStruct((B, E, S), jnp.float32),
        mesh=mesh,
        scratch_types=[
            pltpu.VMEM((IPW, S), jnp.int32),
            pltpu.VMEM((S, E), jnp.float32),
            pltpu.VMEM((S, E), jnp.float32),
            pltpu.VMEM((E, OP), jnp.float32),
            pltpu.VMEM((E, OP), jnp.float32),
            pltpu.SemaphoreType.DMA,
            pltpu.SemaphoreType.DMA,
            pltpu.SemaphoreType.DMA,
            pltpu.SemaphoreType.DMA,
        ],
        compiler_params=pltpu.CompilerParams(
            use_tc_tiling_on_sc=False, needs_layout_passes=False
        ),
    )
    def k(inputs_hbm, table_hbm, out_hbm, idx_v, rows_a, rows_b, out_a, out_b,
          gsem_a, gsem_b, wsem_a, wsem_b):
        wid = lax.axis_index("s") * NC + lax.axis_index("c")
        base_b = wid * IPW

        pltpu.sync_copy(inputs_hbm.at[pl.ds(base_b, IPW), :], idx_v)

        eidx = [jnp.arange(L, dtype=jnp.int32) + j * L for j in range(E // L)]
        rows = (rows_a, rows_b)
        outs = (out_a, out_b)
        gsems = (gsem_a, gsem_b)
        wsems = (wsem_a, wsem_b)

        def start_gather(item, p):
            for (o, n) in chunks:
                pltpu.async_copy(
                    table_hbm.at[idx_v.at[item, pl.ds(o, n)]],
                    rows[p].at[pl.ds(o, n), :],
                    gsems[p],
                )

        def wait_gather(p):
            pltpu.make_async_copy(
                table_hbm.at[pl.ds(0, S), :], rows[p], gsems[p]
            ).wait()

        def start_write(item, p):
            pltpu.async_copy(
                outs[p].at[:, pl.ds(0, S)], out_hbm.at[base_b + item], wsems[p]
            )

        def wait_write(p):
            pltpu.make_async_copy(
                out_hbm.at[0], outs[p].at[:, pl.ds(0, S)], wsems[p]
            ).wait()

        def transpose(p):
            rv, ov = rows[p], outs[p]

            def s_body(i, c):
                s0 = i * SUNROLL
                for u in range(SUNROLL):
                    s = s0 + u
                    sv = jnp.full((L,), s, dtype=jnp.int32)
                    for j in range(E // L):
                        v = rv[s, pl.ds(j * L, L)]
                        plsc.store_scatter(ov, [eidx[j], sv], v)
                return c

            lax.fori_loop(0, S // SUNROLL, s_body, None)

        def process(item, p, it):
            @pl.when(item + 1 < IPW)
            def _():
                start_gather(item + 1, 1 - p)

            wait_gather(p)

            @pl.when(it >= 1)
            def _():
                wait_write(p)

            transpose(p)
            start_write(item, p)

        start_gather(0, 0)

        def pair_body(it, c):
            process(2 * it, 0, it)
            process(2 * it + 1, 1, it)
            return c

        lax.fori_loop(0, IPW // 2, pair_body, None)
        wait_write(0)
        wait_write(1)

    return k


def kernel(inputs, table):
    B, S = inputs.shape
    V, E = table.shape
    inputs = inputs.astype(jnp.int32)
    k = _make_kernel(B, S, V, E)
    return k(inputs, table)
